# baseline (speedup 1.0000x reference)
import jax
import jax.numpy as jnp
from jax.experimental import pallas as pl
from jax.experimental.pallas import tpu as pltpu

N = 4096
HDIM = 128
INDIM = 64
BETA = 0.05
TM = 256
E5 = jnp.float8_e5m2
BF = jnp.bfloat16


def _body(a8_ref, c8_ref, h1a8_ref, h1c8_ref, h1a_ref, h1c_ref,
          ht_ref, xt_ref, wm_ref, bm_ref, wz_ref, bz_ref, wg_ref, bg_ref,
          out_ref):
    ht = ht_ref[...]
    mix = BETA * ht
    h2a = mix + (1.0 - BETA) * jnp.dot(
        a8_ref[...], h1a8_ref[...], preferred_element_type=jnp.float32)
    h2c = mix + (1.0 - BETA) * jnp.dot(
        c8_ref[...], h1c8_ref[...], preferred_element_type=jnp.float32)
    h_cat = jnp.concatenate(
        [ht.astype(BF), h1a_ref[...], h2a.astype(BF),
         h1c_ref[...], h2c.astype(BF)], axis=1)
    h_g = jnp.dot(h_cat, wm_ref[...],
                  preferred_element_type=jnp.float32) + bm_ref[...]
    inp = jnp.concatenate([h_g.astype(BF), xt_ref[...]], axis=1)
    z = jax.nn.sigmoid(
        jnp.dot(inp, wz_ref[...],
                preferred_element_type=jnp.float32) + bz_ref[...])
    g = jnp.tanh(
        jnp.dot(inp, wg_ref[...],
                preferred_element_type=jnp.float32) + bg_ref[...])
    out_ref[...] = z * ht + (1.0 - z) * g


@jax.jit
def kernel(t, H_in, X_in, A, C, W_mlp, b_mlp, W_z, b_z, W_g, b_g):
    del t
    grid = (N // TM,)
    row_tile = pl.BlockSpec((TM, N), lambda i: (i, 0))
    h_tile = pl.BlockSpec((TM, HDIM), lambda i: (i, 0))

    def full(shape):
        return pl.BlockSpec(shape, lambda i: tuple(0 for _ in shape))

    A8 = A.astype(E5)
    C8 = C.astype(E5)
    H1A8 = H_in.astype(E5)
    H1C8 = H_in.astype(E5)
    H1A = H_in.astype(BF)
    H1C = H_in.astype(BF)
    X_bf = X_in.astype(BF)
    out = pl.pallas_call(
        _body,
        grid=grid,
        in_specs=[row_tile, row_tile, full((N, HDIM)), full((N, HDIM)),
                  h_tile, h_tile, h_tile,
                  pl.BlockSpec((TM, INDIM), lambda i: (i, 0)),
                  full((5 * HDIM, HDIM)), full((1, HDIM)),
                  full((HDIM + INDIM, HDIM)), full((1, HDIM)),
                  full((HDIM + INDIM, HDIM)), full((1, HDIM))],
        out_specs=h_tile,
        out_shape=jax.ShapeDtypeStruct((N, HDIM), jnp.float32),
        compiler_params=pltpu.CompilerParams(
            dimension_semantics=("arbitrary",),
            vmem_limit_bytes=100 * 1024 * 1024),
    )(A8, C8, H1A8, H1C8, H1A.astype(BF), H1C, H_in, X_bf,
      W_mlp.astype(BF), b_mlp.reshape(1, HDIM),
      W_z.astype(BF), b_z.reshape(1, HDIM),
      W_g.astype(BF), b_g.reshape(1, HDIM))
    return out


# speedup vs baseline: 1.0151x; 1.0151x over previous
import jax
import jax.numpy as jnp
from jax.experimental import pallas as pl
from jax.experimental.pallas import tpu as pltpu

N = 4096
HDIM = 128
INDIM = 64
BETA = 0.05
TM = 256
W2 = 256
NS = N // TM
E5 = jnp.float8_e5m2
BF = jnp.bfloat16
F32 = jnp.float32


def _nt_dot(lhs, rhs):
    return jax.lax.dot_general(lhs, rhs, (((1,), (1,)), ((), ())),
                               preferred_element_type=F32)


def _body(ct_ref, at_ref, h8t_ref, htt_ref,
          htw_ref, xtw_ref, wm_ref, bm_ref, wz_ref, bz_ref, wg_ref, bg_ref,
          outt_ref, a8_s, c8_s, h1at_s, h1ct_s, h1a8t_s, h1c8t_s, h2ct_s):
    p = pl.program_id(0)
    j = pl.program_id(1)
    rows = pl.ds(j * TM, TM)
    wide = pl.ds(j * W2, W2)

    def hop1(src_ref, ht, s8, s1, s18):
        s = src_ref[...].astype(E5)
        s8[rows, :] = s
        h1t = BETA * ht + (1.0 - BETA) * _nt_dot(h8t_ref[...], s)
        s1[:, rows] = h1t.astype(BF)
        s18[:, rows] = h1t.astype(E5)

    @pl.when(p == 0)
    def _hop1_c():
        hop1(ct_ref, htt_ref[...], c8_s, h1ct_s, h1c8t_s)

    @pl.when(p == 1)
    def _hop1_a_hop2_c():
        hop1(at_ref, htt_ref[...], a8_s, h1at_s, h1a8t_s)
        h2ct = (BETA * htw_ref[...]
                + (1.0 - BETA) * _nt_dot(h1c8t_s[...], c8_s[wide, :]))
        h2ct_s[:, wide] = h2ct.astype(BF)

    @pl.when(p == 2)
    def _hop2_a_epilogue():
        htw = htw_ref[...]
        h2at = (BETA * htw
                + (1.0 - BETA) * _nt_dot(h1a8t_s[...], a8_s[wide, :]))
        h_cat_t = jnp.concatenate(
            [htw.astype(BF), h1at_s[:, wide], h2at.astype(BF),
             h1ct_s[:, wide], h2ct_s[:, wide]], axis=0)
        h_g_t = jax.lax.dot_general(
            wm_ref[...], h_cat_t, (((0,), (0,)), ((), ())),
            preferred_element_type=F32) + bm_ref[...]
        inp_t = jnp.concatenate([h_g_t.astype(BF), xtw_ref[...]], axis=0)
        z = jax.nn.sigmoid(jax.lax.dot_general(
            wz_ref[...], inp_t, (((0,), (0,)), ((), ())),
            preferred_element_type=F32) + bz_ref[...])
        g = jnp.tanh(jax.lax.dot_general(
            wg_ref[...], inp_t, (((0,), (0,)), ((), ())),
            preferred_element_type=F32) + bg_ref[...])
        outt_ref[...] = z * htw + (1.0 - z) * g


@jax.jit
def kernel(t, H_in, X_in, A, C, W_mlp, b_mlp, W_z, b_z, W_g, b_g):
    del t
    grid = (3, NS)
    ct_spec = pl.BlockSpec((TM, N), lambda p, j: (jnp.where(p == 0, j, NS - 1), 0))
    at_spec = pl.BlockSpec((TM, N), lambda p, j: (jnp.where(p == 1, j, 0), 0))

    def full(shape):
        return pl.BlockSpec(shape, lambda p, j: tuple(0 for _ in shape))

    H_t = H_in.T
    H8_t = H_t.astype(E5)
    X_t = X_in.T.astype(BF)
    out_t = pl.pallas_call(
        _body,
        grid=grid,
        in_specs=[ct_spec, at_spec,
                  full((HDIM, N)),
                  pl.BlockSpec((HDIM, TM), lambda p, j: (0, j)),
                  pl.BlockSpec((HDIM, W2),
                               lambda p, j: (0, jnp.where(p == 0, 0, j))),
                  pl.BlockSpec((INDIM, W2),
                               lambda p, j: (0, jnp.where(p == 2, j, 0))),
                  full((5 * HDIM, HDIM)), full((HDIM, 1)),
                  full((HDIM + INDIM, HDIM)), full((HDIM, 1)),
                  full((HDIM + INDIM, HDIM)), full((HDIM, 1))],
        out_specs=pl.BlockSpec((HDIM, W2),
                               lambda p, j: (0, jnp.where(p == 2, j, 0))),
        out_shape=jax.ShapeDtypeStruct((HDIM, N), F32),
        scratch_shapes=[
            pltpu.VMEM((N, N), E5),
            pltpu.VMEM((N, N), E5),
            pltpu.VMEM((HDIM, N), BF),
            pltpu.VMEM((HDIM, N), BF),
            pltpu.VMEM((HDIM, N), E5),
            pltpu.VMEM((HDIM, N), E5),
            pltpu.VMEM((HDIM, N), BF),
        ],
        compiler_params=pltpu.CompilerParams(
            dimension_semantics=("arbitrary", "arbitrary"),
            vmem_limit_bytes=100 * 1024 * 1024),
    )(C, A, H8_t, H_t, H_t, X_t,
      W_mlp.astype(BF), b_mlp.reshape(HDIM, 1),
      W_z.astype(BF), b_z.reshape(HDIM, 1),
      W_g.astype(BF), b_g.reshape(HDIM, 1))
    return out_t.T


# speedup vs baseline: 1.1104x; 1.0939x over previous
import jax
import jax.numpy as jnp
from jax.experimental import pallas as pl
from jax.experimental.pallas import tpu as pltpu

N = 4096
HDIM = 128
INDIM = 64
BETA = 0.05
TM = 128
W2 = 256
NS = N // (2 * TM)
E5 = jnp.float8_e5m2
BF = jnp.bfloat16
F32 = jnp.float32


def _nt_dot(lhs, rhs):
    return jax.lax.dot_general(lhs, rhs, (((1,), (1,)), ((), ())),
                               preferred_element_type=F32)


def _body(ct_ref, cb_ref, at_ref, ab_ref, h8t_ref, htt_ref, htb_ref,
          htw_ref, xtw_ref, wm_ref, bm_ref, wz_ref, bz_ref, wg_ref, bg_ref,
          outt_ref, a8_s, c8_s, h1at_s, h1ct_s, h1a8t_s, h1c8t_s):
    p = pl.program_id(0)
    j = pl.program_id(1)
    top = pl.ds(j * TM, TM)
    bot = pl.ds(N // 2 + j * TM, TM)
    wide = pl.ds(j * W2, W2)

    def hop1(src_ref, rows, ht, s8, s1, s18):
        s = src_ref[...].astype(E5)
        s8[rows, :] = s
        h1t = BETA * ht + (1.0 - BETA) * _nt_dot(h8t_ref[...], s)
        s1[:, rows] = h1t.astype(BF)
        s18[:, rows] = h1t.astype(E5)

    @pl.when(p == 0)
    def _hop1():
        hop1(ct_ref, top, htt_ref[...], c8_s, h1ct_s, h1c8t_s)
        hop1(cb_ref, bot, htb_ref[...], c8_s, h1ct_s, h1c8t_s)
        hop1(at_ref, top, htt_ref[...], a8_s, h1at_s, h1a8t_s)
        hop1(ab_ref, bot, htb_ref[...], a8_s, h1at_s, h1a8t_s)

    @pl.when(p == 1)
    def _hop2_epilogue():
        htw = htw_ref[...]
        h2ct = (BETA * htw
                + (1.0 - BETA) * _nt_dot(h1c8t_s[...], c8_s[wide, :]))
        h2at = (BETA * htw
                + (1.0 - BETA) * _nt_dot(h1a8t_s[...], a8_s[wide, :]))
        h_cat_t = jnp.concatenate(
            [htw.astype(BF), h1at_s[:, wide], h2at.astype(BF),
             h1ct_s[:, wide], h2ct.astype(BF)], axis=0)
        h_g_t = jax.lax.dot_general(
            wm_ref[...], h_cat_t, (((0,), (0,)), ((), ())),
            preferred_element_type=F32) + bm_ref[...]
        inp_t = jnp.concatenate([h_g_t.astype(BF), xtw_ref[...]], axis=0)
        z = jax.nn.sigmoid(jax.lax.dot_general(
            wz_ref[...], inp_t, (((0,), (0,)), ((), ())),
            preferred_element_type=F32) + bz_ref[...])
        g = jnp.tanh(jax.lax.dot_general(
            wg_ref[...], inp_t, (((0,), (0,)), ((), ())),
            preferred_element_type=F32) + bg_ref[...])
        outt_ref[...] = z * htw + (1.0 - z) * g


@jax.jit
def kernel(t, H_in, X_in, A, C, W_mlp, b_mlp, W_z, b_z, W_g, b_g):
    del t
    grid = (2, NS)
    def stream(offset):
        return pl.BlockSpec(
            (TM, N), lambda p, j: (jnp.where(p == 0, j, NS - 1) + offset, 0))

    def full(shape):
        return pl.BlockSpec(shape, lambda p, j: tuple(0 for _ in shape))

    H_t = H_in.T
    H8_t = H_t.astype(E5)
    X_t = X_in.T.astype(BF)
    out_t = pl.pallas_call(
        _body,
        grid=grid,
        in_specs=[stream(0), stream(NS), stream(0), stream(NS),
                  full((HDIM, N)),
                  pl.BlockSpec((HDIM, TM), lambda p, j: (0, j)),
                  pl.BlockSpec((HDIM, TM), lambda p, j: (0, j + NS)),
                  pl.BlockSpec((HDIM, W2),
                               lambda p, j: (0, jnp.where(p == 1, j, 0))),
                  pl.BlockSpec((INDIM, W2),
                               lambda p, j: (0, jnp.where(p == 1, j, 0))),
                  full((5 * HDIM, HDIM)), full((HDIM, 1)),
                  full((HDIM + INDIM, HDIM)), full((HDIM, 1)),
                  full((HDIM + INDIM, HDIM)), full((HDIM, 1))],
        out_specs=pl.BlockSpec((HDIM, W2),
                               lambda p, j: (0, jnp.where(p == 1, j, 0))),
        out_shape=jax.ShapeDtypeStruct((HDIM, N), F32),
        scratch_shapes=[
            pltpu.VMEM((N, N), E5),
            pltpu.VMEM((N, N), E5),
            pltpu.VMEM((HDIM, N), BF),
            pltpu.VMEM((HDIM, N), BF),
            pltpu.VMEM((HDIM, N), E5),
            pltpu.VMEM((HDIM, N), E5),
        ],
        compiler_params=pltpu.CompilerParams(
            dimension_semantics=("arbitrary", "arbitrary"),
            vmem_limit_bytes=100 * 1024 * 1024),
    )(C, C, A, A, H8_t, H_t, H_t, H_t, X_t,
      W_mlp.astype(BF), b_mlp.reshape(HDIM, 1),
      W_z.astype(BF), b_z.reshape(HDIM, 1),
      W_g.astype(BF), b_g.reshape(HDIM, 1))
    return out_t.T


# speedup vs baseline: 1.2184x; 1.0972x over previous
import jax
import jax.numpy as jnp
from jax.experimental import pallas as pl
from jax.experimental.pallas import tpu as pltpu

N = 4096
HDIM = 128
INDIM = 64
BETA = 0.05
TM = 256
E5 = jnp.float8_e5m2
BF = jnp.bfloat16


def _body(a_ref, c_ref, h8_ref, ht_ref, xt_ref,
          wm_ref, bm_ref, wz_ref, bz_ref, wg_ref, bg_ref,
          out_ref, a8_s, c8_s, h1a_s, h1c_s, h1a8_s, h1c8_s):
    p = pl.program_id(0)
    i = pl.program_id(1)
    rows = pl.ds(i * TM, TM)
    ht = ht_ref[...]
    mix = BETA * ht

    @pl.when(p == 0)
    def _hop1():
        a8 = a_ref[...].astype(E5)
        c8 = c_ref[...].astype(E5)
        a8_s[rows, :] = a8
        c8_s[rows, :] = c8
        h8 = h8_ref[...]
        h1a = mix + (1.0 - BETA) * jnp.dot(
            a8, h8, preferred_element_type=jnp.float32)
        h1c = mix + (1.0 - BETA) * jnp.dot(
            c8, h8, preferred_element_type=jnp.float32)
        h1a_s[rows, :] = h1a.astype(BF)
        h1c_s[rows, :] = h1c.astype(BF)
        h1a8_s[rows, :] = h1a.astype(E5)
        h1c8_s[rows, :] = h1c.astype(E5)

    @pl.when(p == 1)
    def _hop2_epilogue():
        h2a = mix + (1.0 - BETA) * jnp.dot(
            a8_s[rows, :], h1a8_s[...], preferred_element_type=jnp.float32)
        h2c = mix + (1.0 - BETA) * jnp.dot(
            c8_s[rows, :], h1c8_s[...], preferred_element_type=jnp.float32)
        h_cat = jnp.concatenate(
            [ht.astype(BF), h1a_s[rows, :], h2a.astype(BF),
             h1c_s[rows, :], h2c.astype(BF)], axis=1)
        h_g = jnp.dot(h_cat, wm_ref[...],
                      preferred_element_type=jnp.float32) + bm_ref[...]
        inp = jnp.concatenate([h_g.astype(BF), xt_ref[...]], axis=1)
        z = jax.nn.sigmoid(
            jnp.dot(inp, wz_ref[...],
                    preferred_element_type=jnp.float32) + bz_ref[...])
        g = jnp.tanh(
            jnp.dot(inp, wg_ref[...],
                    preferred_element_type=jnp.float32) + bg_ref[...])
        out_ref[...] = z * ht + (1.0 - z) * g


@jax.jit
def kernel(t, H_in, X_in, A, C, W_mlp, b_mlp, W_z, b_z, W_g, b_g):
    del t
    grid = (2, N // TM)
    ac_spec = pl.BlockSpec((TM, N), lambda p, i: (i * (1 - p), 0))
    h_tile = pl.BlockSpec((TM, HDIM), lambda p, i: (i, 0))

    def full(shape):
        return pl.BlockSpec(shape, lambda p, i: tuple(0 for _ in shape))

    H8 = H_in.astype(E5)
    X_bf = X_in.astype(BF)
    Wm_bf = W_mlp.astype(BF)
    Wz_bf = W_z.astype(BF)
    Wg_bf = W_g.astype(BF)
    bm2 = b_mlp.reshape(1, HDIM)
    bz2 = b_z.reshape(1, HDIM)
    bg2 = b_g.reshape(1, HDIM)

    out = pl.pallas_call(
        _body,
        grid=grid,
        in_specs=[ac_spec, ac_spec, full((N, HDIM)), h_tile,
                  pl.BlockSpec((TM, INDIM), lambda p, i: (i, 0)),
                  full((5 * HDIM, HDIM)), full((1, HDIM)),
                  full((HDIM + INDIM, HDIM)), full((1, HDIM)),
                  full((HDIM + INDIM, HDIM)), full((1, HDIM))],
        out_specs=pl.BlockSpec((TM, HDIM), lambda p, i: (i * p, 0)),
        out_shape=jax.ShapeDtypeStruct((N, HDIM), jnp.float32),
        scratch_shapes=[
            pltpu.VMEM((N, N), E5),
            pltpu.VMEM((N, N), E5),
            pltpu.VMEM((N, HDIM), BF),
            pltpu.VMEM((N, HDIM), BF),
            pltpu.VMEM((N, HDIM), E5),
            pltpu.VMEM((N, HDIM), E5),
        ],
        compiler_params=pltpu.CompilerParams(
            dimension_semantics=("arbitrary", "arbitrary"),
            vmem_limit_bytes=100 * 1024 * 1024),
    )(A, C, H8, H_in, X_bf, Wm_bf, bm2, Wz_bf, bz2, Wg_bf, bg2)
    return out
